# baseline (speedup 1.0000x reference)
import jax
import jax.numpy as jnp
from jax.experimental import pallas as pl
from jax.experimental.pallas import tpu as pltpu

N = 10000
D_IN = 128
D_OUT = 128
BR = 400
NR = N // BR


def _gcn_kernel(
    feat_ref, weight_ref, bias_ref, adj_hbm, out_ref, buf0, buf1, support_ref, sem0, sem1
):
    r = pl.program_id(0)

    @pl.when(r == 0)
    def _():
        pltpu.make_async_copy(adj_hbm.at[pl.ds(0, BR), :], buf0, sem0).start()
        support_ref[...] = jnp.dot(
            feat_ref[...], weight_ref[...], preferred_element_type=jnp.float32
        )

    nxt = r + 1

    @pl.when((nxt < NR) & (nxt % 2 == 0))
    def _():
        pltpu.make_async_copy(adj_hbm.at[pl.ds(nxt * BR, BR), :], buf0, sem0).start()

    @pl.when((nxt < NR) & (nxt % 2 == 1))
    def _():
        pltpu.make_async_copy(adj_hbm.at[pl.ds(nxt * BR, BR), :], buf1, sem1).start()

    @pl.when(r % 2 == 0)
    def _():
        pltpu.make_async_copy(adj_hbm.at[pl.ds(r * BR, BR), :], buf0, sem0).wait()
        out_ref[...] = (
            jnp.dot(buf0[...], support_ref[...], preferred_element_type=jnp.float32)
            + bias_ref[...]
        )

    @pl.when(r % 2 == 1)
    def _():
        pltpu.make_async_copy(adj_hbm.at[pl.ds(r * BR, BR), :], buf1, sem1).wait()
        out_ref[...] = (
            jnp.dot(buf1[...], support_ref[...], preferred_element_type=jnp.float32)
            + bias_ref[...]
        )


@jax.jit
def kernel(feat, adj, weight, bias):
    bias2d = bias.reshape(1, D_OUT)
    grid = (NR,)
    out = pl.pallas_call(
        _gcn_kernel,
        grid=grid,
        in_specs=[
            pl.BlockSpec((N, D_IN), lambda r: (0, 0)),
            pl.BlockSpec((D_IN, D_OUT), lambda r: (0, 0)),
            pl.BlockSpec((1, D_OUT), lambda r: (0, 0)),
            pl.BlockSpec(memory_space=pl.ANY),
        ],
        out_specs=pl.BlockSpec((BR, D_OUT), lambda r: (r, 0)),
        out_shape=jax.ShapeDtypeStruct((N, D_OUT), jnp.float32),
        scratch_shapes=[
            pltpu.VMEM((BR, N), jnp.float32),
            pltpu.VMEM((BR, N), jnp.float32),
            pltpu.VMEM((N, D_OUT), jnp.float32),
            pltpu.SemaphoreType.DMA,
            pltpu.SemaphoreType.DMA,
        ],
    )(feat, weight, bias2d, adj)
    return out
